# baseline (speedup 1.0000x reference)
import functools

import jax
import jax.numpy as jnp
from jax import lax
from jax.experimental import pallas as pl
from jax.experimental.pallas import tpu as pltpu
from jax.experimental.pallas import tpu_sc as plsc

N = 100000
D = 128
EMB = 128
C = 40
B = 16384
S1 = 5
S2 = 10
ALPHA = 0.2

NC = 2
NS = 16
NW = NC * NS

_MESH = functools.partial(
    plsc.VectorSubcoreMesh, core_axis_name="c", subcore_axis_name="s",
    num_cores=NC, num_subcores=NS)


def _wid():
    return lax.axis_index("s") * NC + lax.axis_index("c")


PA = 80
CH_A = N // PA
ITER_A = -(-CH_A // NW)
IDXA = PA * S1
GA = 5


def _l1_body(neigh_hbm, feat_hbm, out_hbm, idx_v, rows_v, acc_v, sem):
    w = _wid()

    def chunk(it, carry):
        c = w + it * NW

        @pl.when(c < CH_A)
        def _():
            base = c * PA
            pltpu.sync_copy(neigh_hbm.at[pl.ds(base * S1, IDXA)], idx_v)
            pltpu.sync_copy(feat_hbm.at[pl.ds(base, PA)], acc_v)
            g = IDXA // GA
            descs = [
                pltpu.async_copy(
                    feat_hbm.at[idx_v.at[pl.ds(k * g, g)]],
                    rows_v.at[pl.ds(k * g, g)], sem)
                for k in range(GA)
            ]
            for d_ in descs:
                d_.wait()

            def node(i, carry2):
                r0 = i * S1
                for l in range(EMB // 16):
                    sl = pl.ds(l * 16, 16)
                    a = acc_v[i, sl]
                    for j in range(S1):
                        a = a + rows_v[r0 + j, sl]
                    acc_v[i, sl] = a
                return carry2

            lax.fori_loop(0, PA, node, 0)
            pltpu.sync_copy(acc_v, out_hbm.at[pl.ds(base, PA)])

        return carry

    lax.fori_loop(0, ITER_A, chunk, 0)


_l1_sum = pl.kernel(
    _l1_body,
    out_type=jax.ShapeDtypeStruct((N, D), jnp.float32),
    mesh=_MESH(),
    scratch_types=[
        pltpu.VMEM((IDXA,), jnp.int32),
        pltpu.VMEM((IDXA, D), jnp.float32),
        pltpu.VMEM((PA, D), jnp.float32),
        pltpu.SemaphoreType.DMA,
    ],
)


QC = 32
CH_C = B // QC
ITER_C = CH_C // NW
HIDX = QC * 16
GH = 4


def _l2_body(nodes_hbm, tblg_hbm, h1_hbm, out_hbm,
             nv, gidx, grp_v, hidx, rows_v, acc_v, sem):
    w = _wid()

    def chunk(it, carry):
        c = w * ITER_C + it
        base = c * QC
        pltpu.sync_copy(nodes_hbm.at[pl.ds(base, QC)], nv)
        lanes = lax.iota(jnp.int32, 16)
        for t in range(QC // 16):
            nv16 = nv[pl.ds(t * 16, 16)]
            gidx[pl.ds(t * 16, 16)] = jnp.right_shift(nv16, 3)
        pltpu.async_copy(tblg_hbm.at[gidx], grp_v, sem).wait()
        for t in range(QC // 16):
            nv16 = nv[pl.ds(t * 16, 16)]
            for q in range(16):
                i = t * 16 + q
                node = nv16[q]
                vals = grp_v[i, pl.ds(jnp.bitwise_and(node, 7) * 16, 16)]
                vals = jnp.where(lanes == S2, node, vals)
                hidx[pl.ds(i * 16, 16)] = vals
        g = HIDX // GH
        descs = [
            pltpu.async_copy(
                h1_hbm.at[hidx.at[pl.ds(k * g, g)]],
                rows_v.at[pl.ds(k * g, g)], sem)
            for k in range(GH)
        ]
        for d_ in descs:
            d_.wait()

        def seed(i, carry2):
            r0 = i * 16
            for l in range(EMB // 16):
                sl = pl.ds(l * 16, 16)
                a = rows_v[r0, sl]
                for j in range(1, S2 + 1):
                    a = a + rows_v[r0 + j, sl]
                acc_v[i, sl] = a
            return carry2

        lax.fori_loop(0, QC, seed, 0)
        pltpu.sync_copy(acc_v, out_hbm.at[pl.ds(base, QC)])
        return carry

    lax.fori_loop(0, ITER_C, chunk, 0)


_l2_sum = pl.kernel(
    _l2_body,
    out_type=jax.ShapeDtypeStruct((B, EMB), jnp.float32),
    mesh=_MESH(),
    scratch_types=[
        pltpu.VMEM((QC,), jnp.int32),
        pltpu.VMEM((QC,), jnp.int32),
        pltpu.VMEM((QC, 128), jnp.int32),
        pltpu.VMEM((HIDX,), jnp.int32),
        pltpu.VMEM((HIDX, EMB), jnp.float32),
        pltpu.VMEM((QC, EMB), jnp.float32),
        pltpu.SemaphoreType.DMA,
    ],
)


BM1 = 800


def _mm1_body(x_ref, w_ref, o_ref):
    y = jnp.dot(x_ref[...], w_ref[...],
                preferred_element_type=jnp.float32) * (1.0 / (S1 + 1))
    o_ref[...] = jnp.where(y >= 0, y, ALPHA * y)


def _h1_tc(sum1, w1):
    return pl.pallas_call(
        _mm1_body,
        grid=(N // BM1,),
        in_specs=[
            pl.BlockSpec((BM1, D), lambda i: (i, 0)),
            pl.BlockSpec((D, EMB), lambda i: (0, 0)),
        ],
        out_specs=pl.BlockSpec((BM1, EMB), lambda i: (i, 0)),
        out_shape=jax.ShapeDtypeStruct((N, EMB), jnp.float32),
    )(sum1, w1)


BM2 = 1024


def _mm2_body(x_ref, w_ref, cw_ref, o_ref):
    y = jnp.dot(x_ref[...], w_ref[...],
                preferred_element_type=jnp.float32) * (1.0 / (S2 + 1))
    h = jnp.where(y >= 0, y, ALPHA * y)
    o_ref[...] = jnp.dot(h, cw_ref[...], preferred_element_type=jnp.float32)


def _head_tc(sum2, w2, cw_t):
    return pl.pallas_call(
        _mm2_body,
        grid=(B // BM2,),
        in_specs=[
            pl.BlockSpec((BM2, EMB), lambda i: (i, 0)),
            pl.BlockSpec((EMB, EMB), lambda i: (0, 0)),
            pl.BlockSpec((EMB, C), lambda i: (0, 0)),
        ],
        out_specs=pl.BlockSpec((BM2, C), lambda i: (i, 0)),
        out_shape=jax.ShapeDtypeStruct((B, C), jnp.float32),
    )(sum2, w2, cw_t)


def kernel(nodes, neigh_l1, neigh_l2, features, W1, W2, class_weight):
    neigh1_flat = neigh_l1.reshape(-1)
    tblg = jnp.pad(neigh_l2, ((0, 0), (0, 6))).reshape(N // 8, 128)

    sum1 = _l1_sum(neigh1_flat, features)
    h1 = _h1_tc(sum1, W1)
    sum2 = _l2_sum(nodes, tblg, h1)
    return _head_tc(sum2, W2, class_weight.T)
